# Initial kernel scaffold; baseline (speedup 1.0000x reference)
import jax
import jax.numpy as jnp
from jax.experimental import pallas as pl


def kernel(x, W):
    raise NotImplementedError("write your pallas kernel here")


# speedup vs baseline: 1.4758x; 1.4758x over previous
import functools

import jax
import jax.numpy as jnp
from jax import lax
from jax.experimental import pallas as pl
from jax.experimental.pallas import tpu as pltpu
from jax.experimental.pallas import tpu_sc as plsc

VOCAB = 1000000
EMB = 32
BATCH = 4096
HIST = 200
B = BATCH * HIST
NC = 2
NS = 16
NW = NC * NS
BPW = B // NW
CH = 128
NCH = BPW // CH
KG = 10
NOUT = NCH // KG


def _emb_body(x_hbm, w_hbm, out_hbm, idx_v, rows_v, sem):
    wid = lax.axis_index("s") * NC + lax.axis_index("c")
    pltpu.sync_copy(x_hbm.at[pl.ds(wid * NCH, NCH)], idx_v)

    def group(g, carry):
        cps = []
        for j in range(KG):
            cps.append(pltpu.async_copy(
                w_hbm.at[idx_v.at[g * KG + j]],
                rows_v.at[pl.ds(j * CH, CH)],
                sem))
        for cp in cps:
            cp.wait()
        pltpu.sync_copy(
            rows_v,
            out_hbm.at[pl.ds((wid * NCH + g * KG) * CH, KG * CH)])
        return carry

    lax.fori_loop(0, NOUT, group, 0)


@functools.partial(
    pl.kernel,
    out_type=jax.ShapeDtypeStruct((B, EMB), jnp.float32),
    mesh=plsc.VectorSubcoreMesh(core_axis_name="c", subcore_axis_name="s"),
    scratch_types=[
        pltpu.VMEM((NCH, CH), jnp.int32),
        pltpu.VMEM((KG * CH, EMB), jnp.float32),
        pltpu.SemaphoreType.DMA,
    ],
    compiler_params=pltpu.CompilerParams(use_tc_tiling_on_sc=False),
)
def _emb_lookup(x_hbm, w_hbm, out_hbm, idx_v, rows_v, sem):
    _emb_body(x_hbm, w_hbm, out_hbm, idx_v, rows_v, sem)


def _mask_body(x_ref, m_ref):
    m_ref[...] = (x_ref[...] != 0).astype(jnp.float32)


def _mask(x):
    return pl.pallas_call(
        _mask_body,
        out_shape=jax.ShapeDtypeStruct((BATCH, HIST), jnp.float32),
    )(x)


def kernel(x, W):
    x32 = x.astype(jnp.int32)
    emb = _emb_lookup(x32.reshape(B // CH, CH), W)
    mask = _mask(x32)
    return emb.reshape(BATCH, HIST, EMB), mask


# speedup vs baseline: 1.4894x; 1.0092x over previous
import functools

import jax
import jax.numpy as jnp
from jax import lax
from jax.experimental import pallas as pl
from jax.experimental.pallas import tpu as pltpu
from jax.experimental.pallas import tpu_sc as plsc

VOCAB = 1000000
EMB = 32
BATCH = 4096
HIST = 200
B = BATCH * HIST
NC = 2
NS = 16
NW = NC * NS
BPW = B // NW
CH = 128
NCH = BPW // CH
KG = 10
NB = 2
NOUT = NCH // KG
NJ = NOUT // NB


def _emb_body(x_hbm, w_hbm, out_hbm, idx_v, rows_v, gsem0, gsem1, ssem0,
              ssem1):
    wid = lax.axis_index("s") * NC + lax.axis_index("c")
    gsems = (gsem0, gsem1)
    ssems = (ssem0, ssem1)
    pltpu.sync_copy(x_hbm.at[pl.ds(wid * NCH, NCH)], idx_v)

    def body(i, carry):
        cps = []
        for b in range(NB):
            g = i * NB + b
            for j in range(KG):
                cps.append(pltpu.async_copy(
                    w_hbm.at[idx_v.at[g * KG + j]],
                    rows_v.at[pl.ds((b * KG + j) * CH, CH)],
                    gsems[b]))
        sts = []
        for b in range(NB):
            g = i * NB + b
            for cp in cps[b * KG:(b + 1) * KG]:
                cp.wait()
            sts.append(pltpu.async_copy(
                rows_v.at[pl.ds(b * KG * CH, KG * CH)],
                out_hbm.at[pl.ds((wid * NCH + g * KG) * CH, KG * CH)],
                ssems[b]))
        for st in sts:
            st.wait()
        return carry

    lax.fori_loop(0, NJ, body, 0)


@functools.partial(
    pl.kernel,
    out_type=jax.ShapeDtypeStruct((B, EMB), jnp.float32),
    mesh=plsc.VectorSubcoreMesh(core_axis_name="c", subcore_axis_name="s"),
    scratch_types=[
        pltpu.VMEM((NCH, CH), jnp.int32),
        pltpu.VMEM((NB * KG * CH, EMB), jnp.float32),
        pltpu.SemaphoreType.DMA,
        pltpu.SemaphoreType.DMA,
        pltpu.SemaphoreType.DMA,
        pltpu.SemaphoreType.DMA,
    ],
    compiler_params=pltpu.CompilerParams(use_tc_tiling_on_sc=False),
)
def _emb_lookup(x_hbm, w_hbm, out_hbm, idx_v, rows_v, gsem0, gsem1, ssem0,
                ssem1):
    _emb_body(x_hbm, w_hbm, out_hbm, idx_v, rows_v, gsem0, gsem1, ssem0,
              ssem1)


def _mask_body(x_ref, m_ref):
    m_ref[...] = (x_ref[...] != 0).astype(jnp.float32)


def _mask(x):
    return pl.pallas_call(
        _mask_body,
        out_shape=jax.ShapeDtypeStruct((BATCH, HIST), jnp.float32),
    )(x)


def kernel(x, W):
    x32 = x.astype(jnp.int32)
    emb = _emb_lookup(x32.reshape(B // CH, CH), W)
    mask = _mask(x32)
    return emb.reshape(BATCH, HIST, EMB), mask


# speedup vs baseline: 1.4908x; 1.0010x over previous
import functools

import jax
import jax.numpy as jnp
from jax import lax
from jax.experimental import pallas as pl
from jax.experimental.pallas import tpu as pltpu
from jax.experimental.pallas import tpu_sc as plsc

VOCAB = 1000000
EMB = 32
BATCH = 4096
HIST = 200
B = BATCH * HIST
NC = 2
NS = 16
NW = NC * NS
BPW = B // NW
CH = 128
NCH = BPW // CH
KG = 10
NB = 2
NOUT = NCH // KG
NJ = NOUT // NB


def _emb_body(x_hbm, w_hbm, out_hbm, idx_v, rows_v, gsem0, gsem1, ssem0,
              ssem1):
    wid = lax.axis_index("s") * NC + lax.axis_index("c")
    gsems = (gsem0, gsem1)
    ssems = (ssem0, ssem1)
    GSZ = KG * CH
    pltpu.sync_copy(x_hbm.at[pl.ds(wid * BPW, BPW)], idx_v)

    def body(i, carry):
        cps = []
        for b in range(NB):
            g = i * NB + b
            cps.append(pltpu.async_copy(
                w_hbm.at[idx_v.at[pl.ds(g * GSZ, GSZ)]],
                rows_v.at[pl.ds(b * GSZ, GSZ)],
                gsems[b]))
        sts = []
        for b in range(NB):
            g = i * NB + b
            cps[b].wait()
            sts.append(pltpu.async_copy(
                rows_v.at[pl.ds(b * GSZ, GSZ)],
                out_hbm.at[pl.ds(wid * BPW + g * GSZ, GSZ)],
                ssems[b]))
        for st in sts:
            st.wait()
        return carry

    lax.fori_loop(0, NJ, body, 0)


@functools.partial(
    pl.kernel,
    out_type=jax.ShapeDtypeStruct((B, EMB), jnp.float32),
    mesh=plsc.VectorSubcoreMesh(core_axis_name="c", subcore_axis_name="s"),
    scratch_types=[
        pltpu.VMEM((BPW,), jnp.int32),
        pltpu.VMEM((NB * KG * CH, EMB), jnp.float32),
        pltpu.SemaphoreType.DMA,
        pltpu.SemaphoreType.DMA,
        pltpu.SemaphoreType.DMA,
        pltpu.SemaphoreType.DMA,
    ],
    compiler_params=pltpu.CompilerParams(use_tc_tiling_on_sc=False),
)
def _emb_lookup(x_hbm, w_hbm, out_hbm, idx_v, rows_v, gsem0, gsem1, ssem0,
                ssem1):
    _emb_body(x_hbm, w_hbm, out_hbm, idx_v, rows_v, gsem0, gsem1, ssem0,
              ssem1)


def _mask_body(x_ref, m_ref):
    m_ref[...] = (x_ref[...] != 0).astype(jnp.float32)


def _mask(x):
    return pl.pallas_call(
        _mask_body,
        out_shape=jax.ShapeDtypeStruct((BATCH, HIST), jnp.float32),
    )(x)


def kernel(x, W):
    x32 = x.astype(jnp.int32)
    emb = _emb_lookup(x32.reshape(B), W)
    mask = _mask(x32)
    return emb.reshape(BATCH, HIST, EMB), mask


# speedup vs baseline: 1.4910x; 1.0001x over previous
import functools

import jax
import jax.numpy as jnp
from jax import lax
from jax.experimental import pallas as pl
from jax.experimental.pallas import tpu as pltpu
from jax.experimental.pallas import tpu_sc as plsc

VOCAB = 1000000
EMB = 32
BATCH = 4096
HIST = 200
B = BATCH * HIST
NC = 2
NS = 16
NW = NC * NS
BPW = B // NW
CH = 128
NCH = BPW // CH
KG = 5
NB = 4
NOUT = NCH // KG
NJ = NOUT // NB


def _emb_body(x_hbm, w_hbm, out_hbm, idx_v, rows_v, *sems):
    wid = lax.axis_index("s") * NC + lax.axis_index("c")
    gsems = sems[:NB]
    ssems = sems[NB:]
    GSZ = KG * CH
    pltpu.sync_copy(x_hbm.at[pl.ds(wid * BPW, BPW)], idx_v)

    def body(i, carry):
        cps = []
        for b in range(NB):
            g = i * NB + b
            cps.append(pltpu.async_copy(
                w_hbm.at[idx_v.at[pl.ds(g * GSZ, GSZ)]],
                rows_v.at[pl.ds(b * GSZ, GSZ)],
                gsems[b]))
        sts = []
        for b in range(NB):
            g = i * NB + b
            cps[b].wait()
            sts.append(pltpu.async_copy(
                rows_v.at[pl.ds(b * GSZ, GSZ)],
                out_hbm.at[pl.ds(wid * BPW + g * GSZ, GSZ)],
                ssems[b]))
        for st in sts:
            st.wait()
        return carry

    lax.fori_loop(0, NJ, body, 0)


@functools.partial(
    pl.kernel,
    out_type=jax.ShapeDtypeStruct((B, EMB), jnp.float32),
    mesh=plsc.VectorSubcoreMesh(core_axis_name="c", subcore_axis_name="s"),
    scratch_types=[
        pltpu.VMEM((BPW,), jnp.int32),
        pltpu.VMEM((NB * KG * CH, EMB), jnp.float32),
    ] + [pltpu.SemaphoreType.DMA] * (2 * NB),
    compiler_params=pltpu.CompilerParams(use_tc_tiling_on_sc=False),
)
def _emb_lookup(x_hbm, w_hbm, out_hbm, idx_v, rows_v, *sems):
    _emb_body(x_hbm, w_hbm, out_hbm, idx_v, rows_v, *sems)


def _mask_body(x_ref, m_ref):
    m_ref[...] = (x_ref[...] != 0).astype(jnp.float32)


def _mask(x):
    return pl.pallas_call(
        _mask_body,
        out_shape=jax.ShapeDtypeStruct((BATCH, HIST), jnp.float32),
    )(x)


def kernel(x, W):
    x32 = x.astype(jnp.int32)
    emb = _emb_lookup(x32.reshape(B), W)
    mask = _mask(x32)
    return emb.reshape(BATCH, HIST, EMB), mask
